# baseline (speedup 1.0000x reference)
import jax
import jax.numpy as jnp
from jax import lax
from jax.experimental import pallas as pl
from jax.experimental.pallas import tpu as pltpu, tpu_sc as plsc

N_NODES = 10000
N_PAD = 10240
D = 128
E = 320000
B = 8192
NC, NS = 2, 16
NW = NC * NS
CHUNK = 80
NCH0 = 192
NCH1 = 60
TOT_CH = NS * (NCH0 + NCH1)
UNROLL = 6
E_PAD = TOT_CH * CHUNK
NRB = 3
NIB = 6
ROWS_PER_TILE = N_PAD // NS
GCHUNK = 64
SELF_CH = B // NW // GCHUNK
NEIGH_CH = B // NS // GCHUNK


def _sc_body(feats, eidx, nodes, out_self, out_n0, out_n1, out_d0, out_d1,
             agg_sh, deg_sh, r0, r1, r2, i0, i1, i2, i3, i4, i5,
             g0, g1, g2, d0, d1, d2, ones_v, zb,
             sg0, sg1, sg2, ss0, ss1, ss2, sd0, sd1, sd2,
             si0, si1, si2, si3, si4, si5, sw0, sw1, sw2):
    R = (r0, r1, r2)
    I = (i0, i1, i2, i3, i4, i5)
    G = (g0, g1, g2)
    DB = (d0, d1, d2)
    SG = (sg0, sg1, sg2)
    SS = (ss0, ss1, ss2)
    SD = (sd0, sd1, sd2)
    SI = (si0, si1, si2, si3, si4, si5)
    SW = (sw0, sw1, sw2)
    c = lax.axis_index("c")
    s = lax.axis_index("s")
    wid = s * NC + c

    def zrow(i, carry):
        def zcol(j, carry2):
            r0[i, pl.ds(j * 16, 16)] = jnp.zeros((16,), jnp.float32)
            return carry2
        return lax.fori_loop(0, D // 16, zcol, carry)
    lax.fori_loop(0, CHUNK, zrow, 0)

    def zb_fill(i, carry):
        zb[pl.ds(i * 16, 16)] = jnp.zeros((16,), jnp.float32)
        return carry
    lax.fori_loop(0, ROWS_PER_TILE // 16, zb_fill, 0)

    def ones_fill(i, carry):
        ones_v[pl.ds(i * 16, 16)] = jnp.ones((16,), jnp.float32)
        return carry
    lax.fori_loop(0, CHUNK // 16, ones_fill, 0)
    base_rows = s * ROWS_PER_TILE

    def zcopy(k, carry):
        pltpu.sync_copy(r0, agg_sh.at[pl.ds(base_rows + k * CHUNK, CHUNK)])
        return carry
    lax.fori_loop(0, ROWS_PER_TILE // CHUNK, zcopy, 0)
    pltpu.sync_copy(zb, deg_sh.at[pl.ds(base_rows, ROWS_PER_TILE)])
    plsc.subcore_barrier()

    def idx_issue(cid, u):
        pltpu.async_copy(eidx.at[cid], I[u % NIB], SI[u % NIB])

    def idx_wait(u):
        pltpu.make_async_copy(eidx.at[0], I[u % NIB], SI[u % NIB]).wait()

    def gather_issue(u):
        pltpu.async_copy(feats.at[I[u % NIB].at[0]], R[u % NRB], SG[u % NRB])

    def gather_wait(u):
        pltpu.make_async_copy(feats.at[I[u % NIB].at[0]], R[u % NRB],
                              SG[u % NRB]).wait()

    def scatter_issue(u):
        pltpu.async_copy(R[u % NRB], agg_sh.at[I[u % NIB].at[1]],
                         SS[u % NRB], add=True)
        pltpu.async_copy(ones_v, deg_sh.at[I[u % NIB].at[1]],
                         SD[u % NRB], add=True)

    def scatter_wait(u):
        pltpu.make_async_copy(R[u % NRB], agg_sh.at[I[u % NIB].at[1]],
                              SS[u % NRB]).wait()
        pltpu.make_async_copy(ones_v, deg_sh.at[I[u % NIB].at[1]],
                              SD[u % NRB]).wait()

    def run_pipeline(nch, base):
        for u in range(4):
            idx_issue(base + u, u)
        idx_wait(0)
        gather_issue(0)

        def body(t, carry):
            for u in range(UNROLL):
                k = t * UNROLL + u
                if u == 0:
                    @pl.when(t > 0)
                    def _():
                        gather_wait(u - 1)
                        scatter_issue(u - 1)
                else:
                    gather_wait(u - 1)
                    scatter_issue(u - 1)
                def g_issue():
                    idx_wait(u + 1)
                    if u in (0, 1):
                        @pl.when(t > 0)
                        def _():
                            scatter_wait(u + 1)
                    else:
                        scatter_wait(u + 1)
                    gather_issue(u + 1)
                if u == UNROLL - 1:
                    @pl.when(t < nch // UNROLL - 1)
                    def _():
                        g_issue()
                else:
                    g_issue()
                if u in (0, 1):
                    idx_issue(base + k + 4, u + 4)
                else:
                    @pl.when(t < nch // UNROLL - 1)
                    def _():
                        idx_issue(base + k + 4, u + 4)
            return carry
        lax.fori_loop(0, nch // UNROLL, body, 0)

        gather_wait(nch - 1)
        scatter_issue(nch - 1)
        scatter_wait(nch - 3)
        scatter_wait(nch - 2)
        scatter_wait(nch - 1)

    @pl.when(c == 0)
    def _():
        run_pipeline(NCH0, s * NCH0)

    @pl.when(c == 1)
    def _():
        run_pipeline(NCH1, NS * NCH0 + s * NCH1)
    plsc.subcore_barrier()

    sbase = wid * (B // NW)
    nbase = s * (B // NS)
    steps = ([(sbase + ci * GCHUNK, False) for ci in range(SELF_CH)]
             + [(nbase + ci * GCHUNK, True) for ci in range(NEIGH_CH)])
    NSTEP = len(steps)

    def row_out(n, core):
        off, is_neigh = steps[n]
        if is_neigh:
            o = out_n0 if core == 0 else out_n1
            return o.at[pl.ds(off, GCHUNK)]
        return out_self.at[pl.ds(off, GCHUNK)]

    def deg_out(n, core):
        off, _ = steps[n]
        o = out_d0 if core == 0 else out_d1
        return o.at[pl.ds(off, GCHUNK)]

    def p2_idx_issue(n):
        off, _ = steps[n]
        pltpu.async_copy(nodes.at[pl.ds(off, GCHUNK)], G[n % 3], SI[n % 3])

    def p2_idx_wait(n):
        pltpu.make_async_copy(nodes.at[pl.ds(0, GCHUNK)], G[n % 3],
                              SI[n % 3]).wait()

    def p2_gather_issue(n):
        b = n % 3
        src_rows = R[b].at[pl.ds(0, GCHUNK)]
        _, is_neigh = steps[n]
        if is_neigh:
            pltpu.async_copy(agg_sh.at[G[b]], src_rows, SG[b])
            pltpu.async_copy(deg_sh.at[G[b]], DB[b], SW[b])
        else:
            pltpu.async_copy(feats.at[G[b]], src_rows, SG[b])

    def p2_gather_wait(n):
        b = n % 3
        src_rows = R[b].at[pl.ds(0, GCHUNK)]
        _, is_neigh = steps[n]
        if is_neigh:
            pltpu.make_async_copy(agg_sh.at[G[b]], src_rows, SG[b]).wait()
            pltpu.make_async_copy(deg_sh.at[G[b]], DB[b], SW[b]).wait()
        else:
            pltpu.make_async_copy(feats.at[G[b]], src_rows, SG[b]).wait()

    def write_issue(n):
        b = n % 3
        _, is_neigh = steps[n]
        src_rows = R[b].at[pl.ds(0, GCHUNK)]
        @pl.when(c == 0)
        def _():
            pltpu.async_copy(src_rows, row_out(n, 0), SS[b])
        @pl.when(c == 1)
        def _():
            pltpu.async_copy(src_rows, row_out(n, 1), SS[b])
        if is_neigh:
            @pl.when(c == 0)
            def _():
                pltpu.async_copy(DB[b], deg_out(n, 0), SD[b])
            @pl.when(c == 1)
            def _():
                pltpu.async_copy(DB[b], deg_out(n, 1), SD[b])

    def write_wait(n):
        b = n % 3
        _, is_neigh = steps[n]
        src_rows = R[b].at[pl.ds(0, GCHUNK)]
        @pl.when(c == 0)
        def _():
            pltpu.make_async_copy(src_rows, row_out(n, 0), SS[b]).wait()
        @pl.when(c == 1)
        def _():
            pltpu.make_async_copy(src_rows, row_out(n, 1), SS[b]).wait()
        if is_neigh:
            @pl.when(c == 0)
            def _():
                pltpu.make_async_copy(DB[b], deg_out(n, 0), SD[b]).wait()
            @pl.when(c == 1)
            def _():
                pltpu.make_async_copy(DB[b], deg_out(n, 1), SD[b]).wait()

    p2_idx_issue(0)
    for n in range(NSTEP):
        if n >= 3:
            write_wait(n - 3)
        if n + 1 < NSTEP:
            p2_idx_issue(n + 1)
        p2_idx_wait(n)
        p2_gather_issue(n)
        if n >= 1:
            p2_gather_wait(n - 1)
            write_issue(n - 1)
    p2_gather_wait(NSTEP - 1)
    write_issue(NSTEP - 1)
    write_wait(NSTEP - 3)
    write_wait(NSTEP - 2)
    write_wait(NSTEP - 1)


_sc_agg = pl.kernel(
    _sc_body,
    out_type=(
        jax.ShapeDtypeStruct((B, D), jnp.float32),
        jax.ShapeDtypeStruct((B, D), jnp.float32),
        jax.ShapeDtypeStruct((B, D), jnp.float32),
        jax.ShapeDtypeStruct((B,), jnp.float32),
        jax.ShapeDtypeStruct((B,), jnp.float32),
    ),
    mesh=plsc.VectorSubcoreMesh(core_axis_name="c", subcore_axis_name="s",
                                num_cores=NC, num_subcores=NS),
    compiler_params=pltpu.CompilerParams(use_tc_tiling_on_sc=False),
    scratch_types=(
        [pltpu.VMEM_SHARED((N_PAD, D), jnp.float32),
         pltpu.VMEM_SHARED((N_PAD,), jnp.float32)]
        + [pltpu.VMEM((CHUNK, D), jnp.float32) for _ in range(NRB)]
        + [pltpu.VMEM((2, CHUNK), jnp.int32) for _ in range(NIB)]
        + [pltpu.VMEM((GCHUNK,), jnp.int32) for _ in range(3)]
        + [pltpu.VMEM((GCHUNK,), jnp.float32) for _ in range(3)]
        + [pltpu.VMEM((CHUNK,), jnp.float32)]
        + [pltpu.VMEM((ROWS_PER_TILE,), jnp.float32)]
        + [pltpu.SemaphoreType.DMA for _ in range(NRB * 3 + NIB + 3)]
    ),
)

BT = 512


def _tc_body(self_ref, n0_ref, n1_ref, dg0_ref, dg1_ref,
             w1, w2, att, lw, lb, out_ref):
    deg = dg0_ref[...] + dg1_ref[...]
    neigh = (n0_ref[...] + n1_ref[...]) / jnp.maximum(deg, 1.0)
    self_ = self_ref[...]
    dn = (((1,), (1,)), ((), ()))
    h1 = (lax.dot_general(self_, w1[:, :D], dn,
                          preferred_element_type=jnp.float32)
          + lax.dot_general(neigh, w1[:, D:], dn,
                            preferred_element_type=jnp.float32))
    h2 = (lax.dot_general(self_, w2[:, :D], dn,
                          preferred_element_type=jnp.float32)
          + lax.dot_general(neigh, w2[:, D:], dn,
                            preferred_element_type=jnp.float32))
    e1 = jnp.maximum(h1, 0.0)
    e2 = jnp.maximum(h2, 0.0)
    s0 = jnp.dot(e1 * e1, att[...], preferred_element_type=jnp.float32)
    s1 = jnp.dot(e1 * e2, att[...], preferred_element_type=jnp.float32)
    s0 = jnp.where(s0 >= 0.0, s0, 0.5 * s0)
    s1 = jnp.where(s1 >= 0.0, s1, 0.5 * s1)
    m = jnp.maximum(s0, s1)
    a0 = jnp.exp(s0 - m)
    a1 = jnp.exp(s1 - m)
    res = (a0 * e1 + a1 * e2) / (a0 + a1)
    z = jnp.dot(res, lw[...], preferred_element_type=jnp.float32) + lb[0, 0]
    out_ref[...] = 1.0 / (1.0 + jnp.exp(-z))


def _tc_call(out_self, n0, n1, dg0, dg1, w1, w2, att, lw, lb):
    full = lambda i: (0, 0)
    return pl.pallas_call(
        _tc_body,
        grid=(B // BT,),
        in_specs=[
            pl.BlockSpec((BT, D), lambda i: (i, 0)),
            pl.BlockSpec((BT, D), lambda i: (i, 0)),
            pl.BlockSpec((BT, D), lambda i: (i, 0)),
            pl.BlockSpec((BT, 1), lambda i: (i, 0)),
            pl.BlockSpec((BT, 1), lambda i: (i, 0)),
            pl.BlockSpec((D, 2 * D), full),
            pl.BlockSpec((D, 2 * D), full),
            pl.BlockSpec((D, 1), full),
            pl.BlockSpec((D, 1), full),
            pl.BlockSpec((1, 1), full),
        ],
        out_specs=pl.BlockSpec((BT, 1), lambda i: (i, 0)),
        out_shape=jax.ShapeDtypeStruct((B, 1), jnp.float32),
    )(out_self, n0, n1, dg0, dg1, w1, w2, att, lw, lb)


def kernel(nodes, edge_index, features, W1, W2, att_a, logis_W, logis_b):
    npad = E_PAD - E
    padv = jnp.concatenate(
        [jnp.zeros((1, npad), jnp.int32),
         jnp.full((1, npad), N_NODES, jnp.int32)], axis=0)
    eidx = (jnp.concatenate([edge_index, padv], axis=1)
            .reshape(2, TOT_CH, CHUNK).transpose(1, 0, 2))
    out_self, n0, n1, dg0, dg1 = _sc_agg(features, eidx, nodes)
    att = att_a.reshape(D, 1)
    lb = logis_b.reshape(1, 1)
    return _tc_call(out_self, n0, n1, dg0.reshape(B, 1), dg1.reshape(B, 1),
                    W1, W2, att, logis_W, lb)


# speedup vs baseline: 1.0509x; 1.0509x over previous
import jax
import jax.numpy as jnp
from jax import lax
from jax.experimental import pallas as pl
from jax.experimental.pallas import tpu as pltpu, tpu_sc as plsc

N_NODES = 10000
N_PAD = 10240
D = 128
E = 320000
B = 8192
NC, NS = 2, 16
NW = NC * NS
CHUNK = 80
NCH0 = 198
NCH1 = 54
TOT_CH = NS * (NCH0 + NCH1)
UNROLL = 6
E_PAD = TOT_CH * CHUNK
NRB = 3
NIB = 6
ROWS_PER_TILE = N_PAD // NS
GCHUNK = 64
SELF_CH = B // NW // GCHUNK
NEIGH_CH = B // NS // GCHUNK


def _sc_body(feats, eidx, nodes, out_self, out_n0, out_n1, out_d0, out_d1,
             agg_sh, deg_sh, r0, r1, r2, i0, i1, i2, i3, i4, i5,
             g0, g1, g2, d0, d1, d2, ones_v, zb,
             sg0, sg1, sg2, ss0, ss1, ss2, sd0, sd1, sd2,
             si0, si1, si2, si3, si4, si5, sw0, sw1, sw2):
    R = (r0, r1, r2)
    I = (i0, i1, i2, i3, i4, i5)
    G = (g0, g1, g2)
    DB = (d0, d1, d2)
    SG = (sg0, sg1, sg2)
    SS = (ss0, ss1, ss2)
    SD = (sd0, sd1, sd2)
    SI = (si0, si1, si2, si3, si4, si5)
    SW = (sw0, sw1, sw2)
    c = lax.axis_index("c")
    s = lax.axis_index("s")
    wid = s * NC + c

    def zrow(i, carry):
        def zcol(j, carry2):
            r0[i, pl.ds(j * 16, 16)] = jnp.zeros((16,), jnp.float32)
            return carry2
        return lax.fori_loop(0, D // 16, zcol, carry)
    lax.fori_loop(0, CHUNK, zrow, 0)

    def zb_fill(i, carry):
        zb[pl.ds(i * 16, 16)] = jnp.zeros((16,), jnp.float32)
        return carry
    lax.fori_loop(0, ROWS_PER_TILE // 16, zb_fill, 0)

    def ones_fill(i, carry):
        ones_v[pl.ds(i * 16, 16)] = jnp.ones((16,), jnp.float32)
        return carry
    lax.fori_loop(0, CHUNK // 16, ones_fill, 0)
    base_rows = s * ROWS_PER_TILE

    def zcopy(k, carry):
        pltpu.sync_copy(r0, agg_sh.at[pl.ds(base_rows + k * CHUNK, CHUNK)])
        return carry
    lax.fori_loop(0, ROWS_PER_TILE // CHUNK, zcopy, 0)
    pltpu.sync_copy(zb, deg_sh.at[pl.ds(base_rows, ROWS_PER_TILE)])
    plsc.subcore_barrier()

    def idx_issue(cid, u):
        pltpu.async_copy(eidx.at[cid], I[u % NIB], SI[u % NIB])

    def idx_wait(u):
        pltpu.make_async_copy(eidx.at[0], I[u % NIB], SI[u % NIB]).wait()

    def gather_issue(u):
        pltpu.async_copy(feats.at[I[u % NIB].at[0]], R[u % NRB], SG[u % NRB])

    def gather_wait(u):
        pltpu.make_async_copy(feats.at[I[u % NIB].at[0]], R[u % NRB],
                              SG[u % NRB]).wait()

    def scatter_issue(u):
        pltpu.async_copy(R[u % NRB], agg_sh.at[I[u % NIB].at[1]],
                         SS[u % NRB], add=True)
        pltpu.async_copy(ones_v, deg_sh.at[I[u % NIB].at[1]],
                         SD[u % NRB], add=True)

    def scatter_wait(u):
        pltpu.make_async_copy(R[u % NRB], agg_sh.at[I[u % NIB].at[1]],
                              SS[u % NRB]).wait()
        pltpu.make_async_copy(ones_v, deg_sh.at[I[u % NIB].at[1]],
                              SD[u % NRB]).wait()

    def run_pipeline(nch, base):
        for u in range(4):
            idx_issue(base + u, u)
        idx_wait(0)
        gather_issue(0)

        def body(t, carry):
            for u in range(UNROLL):
                k = t * UNROLL + u
                if u == 0:
                    @pl.when(t > 0)
                    def _():
                        gather_wait(u - 1)
                        scatter_issue(u - 1)
                else:
                    gather_wait(u - 1)
                    scatter_issue(u - 1)
                def g_issue():
                    idx_wait(u + 1)
                    if u in (0, 1):
                        @pl.when(t > 0)
                        def _():
                            scatter_wait(u + 1)
                    else:
                        scatter_wait(u + 1)
                    gather_issue(u + 1)
                if u == UNROLL - 1:
                    @pl.when(t < nch // UNROLL - 1)
                    def _():
                        g_issue()
                else:
                    g_issue()
                if u in (0, 1):
                    idx_issue(base + k + 4, u + 4)
                else:
                    @pl.when(t < nch // UNROLL - 1)
                    def _():
                        idx_issue(base + k + 4, u + 4)
            return carry
        lax.fori_loop(0, nch // UNROLL, body, 0)

        gather_wait(nch - 1)
        scatter_issue(nch - 1)
        scatter_wait(nch - 3)
        scatter_wait(nch - 2)
        scatter_wait(nch - 1)

    @pl.when(c == 0)
    def _():
        run_pipeline(NCH0, s * NCH0)

    @pl.when(c == 1)
    def _():
        run_pipeline(NCH1, NS * NCH0 + s * NCH1)
    plsc.subcore_barrier()

    sbase = wid * (B // NW)
    nbase = s * (B // NS)
    steps = ([(sbase + ci * GCHUNK, False) for ci in range(SELF_CH)]
             + [(nbase + ci * GCHUNK, True) for ci in range(NEIGH_CH)])
    NSTEP = len(steps)

    def row_out(n, core):
        off, is_neigh = steps[n]
        if is_neigh:
            o = out_n0 if core == 0 else out_n1
            return o.at[pl.ds(off, GCHUNK)]
        return out_self.at[pl.ds(off, GCHUNK)]

    def deg_out(n, core):
        off, _ = steps[n]
        o = out_d0 if core == 0 else out_d1
        return o.at[pl.ds(off, GCHUNK)]

    def p2_idx_issue(n):
        off, _ = steps[n]
        pltpu.async_copy(nodes.at[pl.ds(off, GCHUNK)], G[n % 3], SI[n % 3])

    def p2_idx_wait(n):
        pltpu.make_async_copy(nodes.at[pl.ds(0, GCHUNK)], G[n % 3],
                              SI[n % 3]).wait()

    def p2_gather_issue(n):
        b = n % 3
        src_rows = R[b].at[pl.ds(0, GCHUNK)]
        _, is_neigh = steps[n]
        if is_neigh:
            pltpu.async_copy(agg_sh.at[G[b]], src_rows, SG[b])
            pltpu.async_copy(deg_sh.at[G[b]], DB[b], SW[b])
        else:
            pltpu.async_copy(feats.at[G[b]], src_rows, SG[b])

    def p2_gather_wait(n):
        b = n % 3
        src_rows = R[b].at[pl.ds(0, GCHUNK)]
        _, is_neigh = steps[n]
        if is_neigh:
            pltpu.make_async_copy(agg_sh.at[G[b]], src_rows, SG[b]).wait()
            pltpu.make_async_copy(deg_sh.at[G[b]], DB[b], SW[b]).wait()
        else:
            pltpu.make_async_copy(feats.at[G[b]], src_rows, SG[b]).wait()

    def write_issue(n):
        b = n % 3
        _, is_neigh = steps[n]
        src_rows = R[b].at[pl.ds(0, GCHUNK)]
        @pl.when(c == 0)
        def _():
            pltpu.async_copy(src_rows, row_out(n, 0), SS[b])
        @pl.when(c == 1)
        def _():
            pltpu.async_copy(src_rows, row_out(n, 1), SS[b])
        if is_neigh:
            @pl.when(c == 0)
            def _():
                pltpu.async_copy(DB[b], deg_out(n, 0), SD[b])
            @pl.when(c == 1)
            def _():
                pltpu.async_copy(DB[b], deg_out(n, 1), SD[b])

    def write_wait(n):
        b = n % 3
        _, is_neigh = steps[n]
        src_rows = R[b].at[pl.ds(0, GCHUNK)]
        @pl.when(c == 0)
        def _():
            pltpu.make_async_copy(src_rows, row_out(n, 0), SS[b]).wait()
        @pl.when(c == 1)
        def _():
            pltpu.make_async_copy(src_rows, row_out(n, 1), SS[b]).wait()
        if is_neigh:
            @pl.when(c == 0)
            def _():
                pltpu.make_async_copy(DB[b], deg_out(n, 0), SD[b]).wait()
            @pl.when(c == 1)
            def _():
                pltpu.make_async_copy(DB[b], deg_out(n, 1), SD[b]).wait()

    p2_idx_issue(0)
    for n in range(NSTEP):
        if n >= 3:
            write_wait(n - 3)
        if n + 1 < NSTEP:
            p2_idx_issue(n + 1)
        p2_idx_wait(n)
        p2_gather_issue(n)
        if n >= 1:
            p2_gather_wait(n - 1)
            write_issue(n - 1)
    p2_gather_wait(NSTEP - 1)
    write_issue(NSTEP - 1)
    write_wait(NSTEP - 3)
    write_wait(NSTEP - 2)
    write_wait(NSTEP - 1)


_sc_agg = pl.kernel(
    _sc_body,
    out_type=(
        jax.ShapeDtypeStruct((B, D), jnp.float32),
        jax.ShapeDtypeStruct((B, D), jnp.float32),
        jax.ShapeDtypeStruct((B, D), jnp.float32),
        jax.ShapeDtypeStruct((B,), jnp.float32),
        jax.ShapeDtypeStruct((B,), jnp.float32),
    ),
    mesh=plsc.VectorSubcoreMesh(core_axis_name="c", subcore_axis_name="s",
                                num_cores=NC, num_subcores=NS),
    compiler_params=pltpu.CompilerParams(use_tc_tiling_on_sc=False),
    scratch_types=(
        [pltpu.VMEM_SHARED((N_PAD, D), jnp.float32),
         pltpu.VMEM_SHARED((N_PAD,), jnp.float32)]
        + [pltpu.VMEM((CHUNK, D), jnp.float32) for _ in range(NRB)]
        + [pltpu.VMEM((2, CHUNK), jnp.int32) for _ in range(NIB)]
        + [pltpu.VMEM((GCHUNK,), jnp.int32) for _ in range(3)]
        + [pltpu.VMEM((GCHUNK,), jnp.float32) for _ in range(3)]
        + [pltpu.VMEM((CHUNK,), jnp.float32)]
        + [pltpu.VMEM((ROWS_PER_TILE,), jnp.float32)]
        + [pltpu.SemaphoreType.DMA for _ in range(NRB * 3 + NIB + 3)]
    ),
)

BT = 512


def _tc_body(self_ref, n0_ref, n1_ref, dg0_ref, dg1_ref,
             w1a, w1b, w2a, w2b, att, lw, lb, out_ref):
    deg = dg0_ref[...] + dg1_ref[...]
    neigh = (n0_ref[...] + n1_ref[...]) / jnp.maximum(deg, 1.0)
    self_ = self_ref[...]
    h1 = (jnp.dot(self_, w1a[...], preferred_element_type=jnp.float32)
          + jnp.dot(neigh, w1b[...], preferred_element_type=jnp.float32))
    h2 = (jnp.dot(self_, w2a[...], preferred_element_type=jnp.float32)
          + jnp.dot(neigh, w2b[...], preferred_element_type=jnp.float32))
    e1 = jnp.maximum(h1, 0.0)
    e2 = jnp.maximum(h2, 0.0)
    s0 = jnp.dot(e1 * e1, att[...], preferred_element_type=jnp.float32)
    s1 = jnp.dot(e1 * e2, att[...], preferred_element_type=jnp.float32)
    s0 = jnp.where(s0 >= 0.0, s0, 0.5 * s0)
    s1 = jnp.where(s1 >= 0.0, s1, 0.5 * s1)
    m = jnp.maximum(s0, s1)
    a0 = jnp.exp(s0 - m)
    a1 = jnp.exp(s1 - m)
    res = (a0 * e1 + a1 * e2) / (a0 + a1)
    z = jnp.dot(res, lw[...], preferred_element_type=jnp.float32) + lb[0, 0]
    out_ref[...] = 1.0 / (1.0 + jnp.exp(-z))


def _tc_call(out_self, n0, n1, dg0, dg1, w1a, w1b, w2a, w2b, att, lw, lb):
    full = lambda i: (0, 0)
    return pl.pallas_call(
        _tc_body,
        grid=(B // BT,),
        in_specs=[
            pl.BlockSpec((BT, D), lambda i: (i, 0)),
            pl.BlockSpec((BT, D), lambda i: (i, 0)),
            pl.BlockSpec((BT, D), lambda i: (i, 0)),
            pl.BlockSpec((BT, 1), lambda i: (i, 0)),
            pl.BlockSpec((BT, 1), lambda i: (i, 0)),
            pl.BlockSpec((D, D), full),
            pl.BlockSpec((D, D), full),
            pl.BlockSpec((D, D), full),
            pl.BlockSpec((D, D), full),
            pl.BlockSpec((D, 1), full),
            pl.BlockSpec((D, 1), full),
            pl.BlockSpec((1, 1), full),
        ],
        out_specs=pl.BlockSpec((BT, 1), lambda i: (i, 0)),
        out_shape=jax.ShapeDtypeStruct((B, 1), jnp.float32),
    )(out_self, n0, n1, dg0, dg1, w1a, w1b, w2a, w2b, att, lw, lb)


def kernel(nodes, edge_index, features, W1, W2, att_a, logis_W, logis_b):
    src = edge_index[0]
    dst = edge_index[1]
    npad = E_PAD - E
    src_p = jnp.concatenate([src, jnp.zeros((npad,), jnp.int32)])
    dst_p = jnp.concatenate([dst, jnp.full((npad,), N_NODES, jnp.int32)])
    eidx = jnp.stack([src_p.reshape(TOT_CH, CHUNK),
                      dst_p.reshape(TOT_CH, CHUNK)], axis=1)
    out_self, n0, n1, dg0, dg1 = _sc_agg(features, eidx, nodes)
    w1a = W1[:, :D].T
    w1b = W1[:, D:].T
    w2a = W2[:, :D].T
    w2b = W2[:, D:].T
    att = att_a.reshape(D, 1)
    lb = logis_b.reshape(1, 1)
    return _tc_call(out_self, n0, n1, dg0.reshape(B, 1), dg1.reshape(B, 1),
                    w1a, w1b, w2a, w2b, att, logis_W, lb)


# speedup vs baseline: 1.0607x; 1.0093x over previous
import jax
import jax.numpy as jnp
from jax import lax
from jax.experimental import pallas as pl
from jax.experimental.pallas import tpu as pltpu, tpu_sc as plsc

N_NODES = 10000
N_PAD = 10240
D = 128
E = 320000
B = 8192
NC, NS = 2, 16
NW = NC * NS
CHUNK = 80
NCH0 = 204
NCH1 = 48
TOT_CH = NS * (NCH0 + NCH1)
UNROLL = 6
E_PAD = TOT_CH * CHUNK
NRB = 3
NIB = 6
ROWS_PER_TILE = N_PAD // NS
GCHUNK = 64
SELF_CH = B // NW // GCHUNK
NEIGH_CH = B // NS // GCHUNK


def _sc_body(feats, eidx, nodes, out_self, out_n0, out_n1, out_d0, out_d1,
             agg_sh, deg_sh, r0, r1, r2, i0, i1, i2, i3, i4, i5,
             g0, g1, g2, d0, d1, d2, ones_v, zb,
             sg0, sg1, sg2, ss0, ss1, ss2, sd0, sd1, sd2,
             si0, si1, si2, si3, si4, si5, sw0, sw1, sw2):
    R = (r0, r1, r2)
    I = (i0, i1, i2, i3, i4, i5)
    G = (g0, g1, g2)
    DB = (d0, d1, d2)
    SG = (sg0, sg1, sg2)
    SS = (ss0, ss1, ss2)
    SD = (sd0, sd1, sd2)
    SI = (si0, si1, si2, si3, si4, si5)
    SW = (sw0, sw1, sw2)
    c = lax.axis_index("c")
    s = lax.axis_index("s")
    wid = s * NC + c

    def zrow(i, carry):
        def zcol(j, carry2):
            r0[i, pl.ds(j * 16, 16)] = jnp.zeros((16,), jnp.float32)
            return carry2
        return lax.fori_loop(0, D // 16, zcol, carry)
    lax.fori_loop(0, CHUNK, zrow, 0)

    def zb_fill(i, carry):
        zb[pl.ds(i * 16, 16)] = jnp.zeros((16,), jnp.float32)
        return carry
    lax.fori_loop(0, ROWS_PER_TILE // 16, zb_fill, 0)

    def ones_fill(i, carry):
        ones_v[pl.ds(i * 16, 16)] = jnp.ones((16,), jnp.float32)
        return carry
    lax.fori_loop(0, CHUNK // 16, ones_fill, 0)
    base_rows = s * ROWS_PER_TILE

    def zcopy(k, carry):
        pltpu.sync_copy(r0, agg_sh.at[pl.ds(base_rows + k * CHUNK, CHUNK)])
        return carry
    lax.fori_loop(0, ROWS_PER_TILE // CHUNK, zcopy, 0)
    pltpu.sync_copy(zb, deg_sh.at[pl.ds(base_rows, ROWS_PER_TILE)])
    plsc.subcore_barrier()

    def idx_issue(cid, u):
        pltpu.async_copy(eidx.at[cid], I[u % NIB], SI[u % NIB])

    def idx_wait(u):
        pltpu.make_async_copy(eidx.at[0], I[u % NIB], SI[u % NIB]).wait()

    def gather_issue(u):
        pltpu.async_copy(feats.at[I[u % NIB].at[0]], R[u % NRB], SG[u % NRB])

    def gather_wait(u):
        pltpu.make_async_copy(feats.at[I[u % NIB].at[0]], R[u % NRB],
                              SG[u % NRB]).wait()

    def scatter_issue(u):
        pltpu.async_copy(R[u % NRB], agg_sh.at[I[u % NIB].at[1]],
                         SS[u % NRB], add=True)
        pltpu.async_copy(ones_v, deg_sh.at[I[u % NIB].at[1]],
                         SD[u % NRB], add=True)

    def scatter_wait(u):
        pltpu.make_async_copy(R[u % NRB], agg_sh.at[I[u % NIB].at[1]],
                              SS[u % NRB]).wait()
        pltpu.make_async_copy(ones_v, deg_sh.at[I[u % NIB].at[1]],
                              SD[u % NRB]).wait()

    def run_pipeline(nch, base):
        for u in range(4):
            idx_issue(base + u, u)
        idx_wait(0)
        gather_issue(0)

        def body(t, carry):
            for u in range(UNROLL):
                k = t * UNROLL + u
                if u == 0:
                    @pl.when(t > 0)
                    def _():
                        gather_wait(u - 1)
                        scatter_issue(u - 1)
                else:
                    gather_wait(u - 1)
                    scatter_issue(u - 1)
                def g_issue():
                    idx_wait(u + 1)
                    if u in (0, 1):
                        @pl.when(t > 0)
                        def _():
                            scatter_wait(u + 1)
                    else:
                        scatter_wait(u + 1)
                    gather_issue(u + 1)
                if u == UNROLL - 1:
                    @pl.when(t < nch // UNROLL - 1)
                    def _():
                        g_issue()
                else:
                    g_issue()
                if u in (0, 1):
                    idx_issue(base + k + 4, u + 4)
                else:
                    @pl.when(t < nch // UNROLL - 1)
                    def _():
                        idx_issue(base + k + 4, u + 4)
            return carry
        lax.fori_loop(0, nch // UNROLL, body, 0)

        gather_wait(nch - 1)
        scatter_issue(nch - 1)
        scatter_wait(nch - 3)
        scatter_wait(nch - 2)
        scatter_wait(nch - 1)

    @pl.when(c == 0)
    def _():
        run_pipeline(NCH0, s * NCH0)

    @pl.when(c == 1)
    def _():
        run_pipeline(NCH1, NS * NCH0 + s * NCH1)
    plsc.subcore_barrier()

    sbase = wid * (B // NW)
    nbase = s * (B // NS)
    steps = ([(sbase + ci * GCHUNK, False) for ci in range(SELF_CH)]
             + [(nbase + ci * GCHUNK, True) for ci in range(NEIGH_CH)])
    NSTEP = len(steps)

    def row_out(n, core):
        off, is_neigh = steps[n]
        if is_neigh:
            o = out_n0 if core == 0 else out_n1
            return o.at[pl.ds(off, GCHUNK)]
        return out_self.at[pl.ds(off, GCHUNK)]

    def deg_out(n, core):
        off, _ = steps[n]
        o = out_d0 if core == 0 else out_d1
        return o.at[pl.ds(off, GCHUNK)]

    def p2_idx_issue(n):
        off, _ = steps[n]
        pltpu.async_copy(nodes.at[pl.ds(off, GCHUNK)], G[n % 3], SI[n % 3])

    def p2_idx_wait(n):
        pltpu.make_async_copy(nodes.at[pl.ds(0, GCHUNK)], G[n % 3],
                              SI[n % 3]).wait()

    def p2_gather_issue(n):
        b = n % 3
        src_rows = R[b].at[pl.ds(0, GCHUNK)]
        _, is_neigh = steps[n]
        if is_neigh:
            pltpu.async_copy(agg_sh.at[G[b]], src_rows, SG[b])
            pltpu.async_copy(deg_sh.at[G[b]], DB[b], SW[b])
        else:
            pltpu.async_copy(feats.at[G[b]], src_rows, SG[b])

    def p2_gather_wait(n):
        b = n % 3
        src_rows = R[b].at[pl.ds(0, GCHUNK)]
        _, is_neigh = steps[n]
        if is_neigh:
            pltpu.make_async_copy(agg_sh.at[G[b]], src_rows, SG[b]).wait()
            pltpu.make_async_copy(deg_sh.at[G[b]], DB[b], SW[b]).wait()
        else:
            pltpu.make_async_copy(feats.at[G[b]], src_rows, SG[b]).wait()

    def write_issue(n):
        b = n % 3
        _, is_neigh = steps[n]
        src_rows = R[b].at[pl.ds(0, GCHUNK)]
        @pl.when(c == 0)
        def _():
            pltpu.async_copy(src_rows, row_out(n, 0), SS[b])
        @pl.when(c == 1)
        def _():
            pltpu.async_copy(src_rows, row_out(n, 1), SS[b])
        if is_neigh:
            @pl.when(c == 0)
            def _():
                pltpu.async_copy(DB[b], deg_out(n, 0), SD[b])
            @pl.when(c == 1)
            def _():
                pltpu.async_copy(DB[b], deg_out(n, 1), SD[b])

    def write_wait(n):
        b = n % 3
        _, is_neigh = steps[n]
        src_rows = R[b].at[pl.ds(0, GCHUNK)]
        @pl.when(c == 0)
        def _():
            pltpu.make_async_copy(src_rows, row_out(n, 0), SS[b]).wait()
        @pl.when(c == 1)
        def _():
            pltpu.make_async_copy(src_rows, row_out(n, 1), SS[b]).wait()
        if is_neigh:
            @pl.when(c == 0)
            def _():
                pltpu.make_async_copy(DB[b], deg_out(n, 0), SD[b]).wait()
            @pl.when(c == 1)
            def _():
                pltpu.make_async_copy(DB[b], deg_out(n, 1), SD[b]).wait()

    p2_idx_issue(0)
    for n in range(NSTEP):
        if n >= 3:
            write_wait(n - 3)
        if n + 1 < NSTEP:
            p2_idx_issue(n + 1)
        p2_idx_wait(n)
        p2_gather_issue(n)
        if n >= 1:
            p2_gather_wait(n - 1)
            write_issue(n - 1)
    p2_gather_wait(NSTEP - 1)
    write_issue(NSTEP - 1)
    write_wait(NSTEP - 3)
    write_wait(NSTEP - 2)
    write_wait(NSTEP - 1)


_sc_agg = pl.kernel(
    _sc_body,
    out_type=(
        jax.ShapeDtypeStruct((B, D), jnp.float32),
        jax.ShapeDtypeStruct((B, D), jnp.float32),
        jax.ShapeDtypeStruct((B, D), jnp.float32),
        jax.ShapeDtypeStruct((B,), jnp.float32),
        jax.ShapeDtypeStruct((B,), jnp.float32),
    ),
    mesh=plsc.VectorSubcoreMesh(core_axis_name="c", subcore_axis_name="s",
                                num_cores=NC, num_subcores=NS),
    compiler_params=pltpu.CompilerParams(use_tc_tiling_on_sc=False),
    scratch_types=(
        [pltpu.VMEM_SHARED((N_PAD, D), jnp.float32),
         pltpu.VMEM_SHARED((N_PAD,), jnp.float32)]
        + [pltpu.VMEM((CHUNK, D), jnp.float32) for _ in range(NRB)]
        + [pltpu.VMEM((2, CHUNK), jnp.int32) for _ in range(NIB)]
        + [pltpu.VMEM((GCHUNK,), jnp.int32) for _ in range(3)]
        + [pltpu.VMEM((GCHUNK,), jnp.float32) for _ in range(3)]
        + [pltpu.VMEM((CHUNK,), jnp.float32)]
        + [pltpu.VMEM((ROWS_PER_TILE,), jnp.float32)]
        + [pltpu.SemaphoreType.DMA for _ in range(NRB * 3 + NIB + 3)]
    ),
)

BT = 512


def _tc_body(self_ref, n0_ref, n1_ref, dg0_ref, dg1_ref,
             w1a, w1b, w2a, w2b, att, lw, lb, out_ref):
    deg = dg0_ref[...] + dg1_ref[...]
    neigh = (n0_ref[...] + n1_ref[...]) / jnp.maximum(deg, 1.0)
    self_ = self_ref[...]
    h1 = (jnp.dot(self_, w1a[...], preferred_element_type=jnp.float32)
          + jnp.dot(neigh, w1b[...], preferred_element_type=jnp.float32))
    h2 = (jnp.dot(self_, w2a[...], preferred_element_type=jnp.float32)
          + jnp.dot(neigh, w2b[...], preferred_element_type=jnp.float32))
    e1 = jnp.maximum(h1, 0.0)
    e2 = jnp.maximum(h2, 0.0)
    s0 = jnp.dot(e1 * e1, att[...], preferred_element_type=jnp.float32)
    s1 = jnp.dot(e1 * e2, att[...], preferred_element_type=jnp.float32)
    s0 = jnp.where(s0 >= 0.0, s0, 0.5 * s0)
    s1 = jnp.where(s1 >= 0.0, s1, 0.5 * s1)
    m = jnp.maximum(s0, s1)
    a0 = jnp.exp(s0 - m)
    a1 = jnp.exp(s1 - m)
    res = (a0 * e1 + a1 * e2) / (a0 + a1)
    z = jnp.dot(res, lw[...], preferred_element_type=jnp.float32) + lb[0, 0]
    out_ref[...] = 1.0 / (1.0 + jnp.exp(-z))


def _tc_call(out_self, n0, n1, dg0, dg1, w1a, w1b, w2a, w2b, att, lw, lb):
    full = lambda i: (0, 0)
    return pl.pallas_call(
        _tc_body,
        grid=(B // BT,),
        in_specs=[
            pl.BlockSpec((BT, D), lambda i: (i, 0)),
            pl.BlockSpec((BT, D), lambda i: (i, 0)),
            pl.BlockSpec((BT, D), lambda i: (i, 0)),
            pl.BlockSpec((BT, 1), lambda i: (i, 0)),
            pl.BlockSpec((BT, 1), lambda i: (i, 0)),
            pl.BlockSpec((D, D), full),
            pl.BlockSpec((D, D), full),
            pl.BlockSpec((D, D), full),
            pl.BlockSpec((D, D), full),
            pl.BlockSpec((D, 1), full),
            pl.BlockSpec((D, 1), full),
            pl.BlockSpec((1, 1), full),
        ],
        out_specs=pl.BlockSpec((BT, 1), lambda i: (i, 0)),
        out_shape=jax.ShapeDtypeStruct((B, 1), jnp.float32),
    )(out_self, n0, n1, dg0, dg1, w1a, w1b, w2a, w2b, att, lw, lb)


def kernel(nodes, edge_index, features, W1, W2, att_a, logis_W, logis_b):
    src = edge_index[0]
    dst = edge_index[1]
    npad = E_PAD - E
    src_p = jnp.concatenate([src, jnp.zeros((npad,), jnp.int32)])
    dst_p = jnp.concatenate([dst, jnp.full((npad,), N_NODES, jnp.int32)])
    eidx = jnp.stack([src_p.reshape(TOT_CH, CHUNK),
                      dst_p.reshape(TOT_CH, CHUNK)], axis=1)
    out_self, n0, n1, dg0, dg1 = _sc_agg(features, eidx, nodes)
    w1a = W1[:, :D].T
    w1b = W1[:, D:].T
    w2a = W2[:, :D].T
    w2b = W2[:, D:].T
    att = att_a.reshape(D, 1)
    lb = logis_b.reshape(1, 1)
    return _tc_call(out_self, n0, n1, dg0.reshape(B, 1), dg1.reshape(B, 1),
                    w1a, w1b, w2a, w2b, att, logis_W, lb)
